# Initial kernel scaffold; baseline (speedup 1.0000x reference)
import jax
import jax.numpy as jnp
from jax.experimental import pallas as pl


def kernel(x):
    raise NotImplementedError("write your pallas kernel here")


# speedup vs baseline: 5.6070x; 5.6070x over previous
import jax
import jax.numpy as jnp
from jax.experimental import pallas as pl

EPS_ = 0.1


def _zono_body(x_ref, o_ref):
    xv = x_ref[0]
    lo = xv < EPS_
    hi = xv > 1.0 - EPS_
    center = jnp.where(lo, (xv + EPS_) * 0.5,
             jnp.where(hi, (xv + 1.0 - EPS_) * 0.5, xv))
    err = jnp.where(lo, (EPS_ + xv) * 0.5,
          jnp.where(hi, (1.0 - xv + EPS_) * 0.5, jnp.full_like(xv, EPS_)))
    E, P = o_ref.shape[1], o_ref.shape[2]
    rows = jax.lax.broadcasted_iota(jnp.int32, (E, P), 0)
    cols = jax.lax.broadcasted_iota(jnp.int32, (E, P), 1)
    out = jnp.where(rows == 0, center,
          jnp.where(rows == cols + 1, err, 0.0))
    o_ref[0] = out


def kernel(x):
    B, C, H, W = x.shape
    P = C * H * W
    E = 1 + P
    x2 = x.reshape(B, 1, P)
    out = pl.pallas_call(
        _zono_body,
        grid=(B,),
        in_specs=[pl.BlockSpec((1, 1, P), lambda b: (b, 0, 0))],
        out_specs=pl.BlockSpec((1, E, P), lambda b: (b, 0, 0)),
        out_shape=jax.ShapeDtypeStruct((B, E, P), x.dtype),
    )(x2)
    return out.reshape(B, E, C, H, W)


# speedup vs baseline: 9.5676x; 1.7064x over previous
import jax
import jax.numpy as jnp
from jax.experimental import pallas as pl

EPS_ = 0.1


def _zono_body(x_ref, o_ref):
    hwb = x_ref.shape[1]
    e_dim = o_ref.shape[3]
    hw0 = pl.program_id(1) * hwb
    xv = x_ref[0]
    lo = xv < EPS_
    hi = xv > 1.0 - EPS_
    center = jnp.where(lo, (xv + EPS_) * 0.5,
             jnp.where(hi, (xv + 1.0 - EPS_) * 0.5, xv))
    err = jnp.where(lo, (EPS_ + xv) * 0.5,
          jnp.where(hi, (1.0 - xv + EPS_) * 0.5, jnp.full_like(xv, EPS_)))
    r = jax.lax.broadcasted_iota(jnp.int32, (hwb, e_dim), 0)
    e = jax.lax.broadcasted_iota(jnp.int32, (hwb, e_dim), 1)
    val = jnp.where(e == 0, center,
          jnp.where(e == r + (hw0 + 1), err, 0.0))
    o_ref[0, :, 0, :] = val


def kernel(x):
    B, C, H, W = x.shape
    P = C * H * W
    E = 1 + P
    HWB = 112
    x3 = x.reshape(B, P, 1)
    out4 = pl.pallas_call(
        _zono_body,
        grid=(B, P // HWB),
        in_specs=[pl.BlockSpec((1, HWB, 1), lambda b, j: (b, j, 0))],
        out_specs=pl.BlockSpec((1, HWB, 1, E), lambda b, j: (b, j, 0, 0)),
        out_shape=jax.ShapeDtypeStruct((B, P, 1, E), x.dtype),
    )(x3)
    return out4.reshape(B, H, W, 1, E).transpose(0, 4, 3, 1, 2)


# speedup vs baseline: 26.0665x; 2.7244x over previous
import jax
import jax.numpy as jnp
from jax.experimental import pallas as pl

EPS_ = 0.1


def _zono_body(x_ref, o_ref):
    hwb = x_ref.shape[1]
    e_dim = o_ref.shape[3]
    hw0 = pl.program_id(1) * hwb
    xv = x_ref[0]
    lo = xv < EPS_
    hi = xv > 1.0 - EPS_
    center = jnp.where(lo, (xv + EPS_) * 0.5,
             jnp.where(hi, (xv + 1.0 - EPS_) * 0.5, xv))
    err = jnp.where(lo, (EPS_ + xv) * 0.5,
          jnp.where(hi, (1.0 - xv + EPS_) * 0.5, jnp.full_like(xv, EPS_)))
    r = jax.lax.broadcasted_iota(jnp.int32, (hwb, e_dim), 0)
    e = jax.lax.broadcasted_iota(jnp.int32, (hwb, e_dim), 1)
    val = jnp.where(e == 0, center,
          jnp.where(e == r + (hw0 + 1), err, 0.0))
    o_ref[0, :, 0, :] = val


def kernel(x):
    B, C, H, W = x.shape
    P = C * H * W
    E = 1 + P
    HWB = 784
    x3 = x.reshape(B, P, 1)
    out4 = pl.pallas_call(
        _zono_body,
        grid=(B, P // HWB),
        in_specs=[pl.BlockSpec((1, HWB, 1), lambda b, j: (b, j, 0))],
        out_specs=pl.BlockSpec((1, HWB, 1, E), lambda b, j: (b, j, 0, 0)),
        out_shape=jax.ShapeDtypeStruct((B, P, 1, E), x.dtype),
    )(x3)
    return out4.reshape(B, H, W, 1, E).transpose(0, 4, 3, 1, 2)


# speedup vs baseline: 31.9153x; 1.2244x over previous
import jax
import jax.numpy as jnp
from jax.experimental import pallas as pl

EPS_ = 0.1


def _zono_body(x_ref, o_ref):
    bb, hwb = x_ref.shape[0], x_ref.shape[1]
    e_dim = o_ref.shape[3]
    xv = x_ref[:, :, 0:1]
    lo = xv < EPS_
    hi = xv > 1.0 - EPS_
    center = jnp.where(lo, (xv + EPS_) * 0.5,
             jnp.where(hi, (xv + 1.0 - EPS_) * 0.5, xv))
    err = jnp.where(lo, (EPS_ + xv) * 0.5,
          jnp.where(hi, (1.0 - xv + EPS_) * 0.5, jnp.full_like(xv, EPS_)))
    r = jax.lax.broadcasted_iota(jnp.int32, (bb, hwb, e_dim), 1)
    e = jax.lax.broadcasted_iota(jnp.int32, (bb, hwb, e_dim), 2)
    val = jnp.where(e == 0, center,
          jnp.where(e == r + 1, err, 0.0))
    o_ref[:, :, 0, :] = val


def kernel(x):
    B, C, H, W = x.shape
    P = C * H * W
    E = 1 + P
    BB = 4
    x3 = x.reshape(B, P, 1)
    out4 = pl.pallas_call(
        _zono_body,
        grid=(B // BB,),
        in_specs=[pl.BlockSpec((BB, P, 1), lambda b: (b, 0, 0))],
        out_specs=pl.BlockSpec((BB, P, 1, E), lambda b: (b, 0, 0, 0)),
        out_shape=jax.ShapeDtypeStruct((B, P, 1, E), x.dtype),
    )(x3)
    return out4.reshape(B, H, W, 1, E).transpose(0, 4, 3, 1, 2)
